# baseline (speedup 1.0000x reference)
import functools

import jax
import jax.numpy as jnp
from jax import lax
from jax.experimental import pallas as pl
from jax.experimental.pallas import tpu as pltpu
from jax.experimental.pallas import tpu_sc as plsc

M = 1_000_000
D = 32
B_UPD = 16_384
B_LKP = 4_096
NF = 26

NC, NS, L = 2, 16, 16
NW = NC * NS
CH = 62_528
M_PAD = NS * CH
BPW = B_LKP // NW
LPW = BPW * NF
GRP = 4 * NF
NGRP = LPW // GRP
NBLK = BPW // L
GRPH = 112
NVH = (LPW + GRPH) // L

_mesh = plsc.VectorSubcoreMesh(core_axis_name="c", subcore_axis_name="s")
_params = pltpu.CompilerParams(needs_layout_passes=False,
                               use_tc_tiling_on_sc=False)


@functools.partial(
    pl.kernel,
    out_type=jax.ShapeDtypeStruct((B_LKP, D), jnp.float32),
    mesh=_mesh,
    scratch_types=[
        pltpu.VMEM((B_UPD,), jnp.int32),
        pltpu.VMEM((CH,), jnp.int32),
        pltpu.VMEM((LPW,), jnp.int32),
        pltpu.VMEM((BPW, NF), jnp.int32),
        pltpu.VMEM((LPW + L,), jnp.int32),
        pltpu.VMEM((2, LPW), jnp.float32),
        pltpu.VMEM((BPW, D), jnp.float32),
        pltpu.VMEM((NVH * L,), jnp.int32),
        pltpu.VMEM((NVH * L,), jnp.int32),
        pltpu.VMEM((NVH * L,), jnp.int32),
        pltpu.VMEM((D, GRPH), jnp.float32),
        pltpu.VMEM((D, GRPH), jnp.float32),
        pltpu.HBM((M_PAD,), jnp.int32),
        pltpu.SemaphoreType.DMA,
        pltpu.SemaphoreType.DMA,
        pltpu.SemaphoreType.DMA,
    ],
    compiler_params=_params,
)
def _fused(memt_hbm, idx_hbm, valt_hbm, lidx_hbm, out_hbm,
           idx_v, own_v, lidx_v, lidx2d, o_v, gbuf, acc, hslot, ho, hlidx,
           hv, hm, owner_hbm, sem_o, sem_m, sem_h):
    s = lax.axis_index("s")
    w = s * NC + lax.axis_index("c")
    iota = lax.iota(jnp.int32, L)
    pltpu.sync_copy(lidx_hbm.at[pl.ds(w * BPW, BPW)], lidx2d)

    def flat(i, c):
        base = i * L
        fi = iota + base
        lidx_v[pl.ds(base, L)] = plsc.load_gather(lidx2d, [fi // NF, fi % NF])
        return c

    lax.fori_loop(0, LPW // L, flat, 0)

    def start_plane(c, par):
        pltpu.make_async_copy(
            memt_hbm.at[c].at[lidx_v], gbuf.at[par], sem_m).start()

    def wait_plane(par):
        pltpu.make_async_copy(
            memt_hbm.at[0].at[lidx_v], gbuf.at[par], sem_m).wait()

    start_plane(0, 0)
    start_plane(1, 1)

    lo = s * CH
    pltpu.sync_copy(idx_hbm, idx_v)
    neg = jnp.full((L,), -1, jnp.int32)

    def mset(i, c):
        own_v[pl.ds(i * L, L)] = neg
        return c

    lax.fori_loop(0, CH // L, mset, 0)

    def scan(j, c):
        base = j * L
        rel = idx_v[pl.ds(base, L)] - lo
        m = (rel >= 0) & (rel < CH)
        plsc.store_scatter(own_v, [rel], iota + base, mask=m)
        return c

    lax.fori_loop(0, B_UPD // L, scan, 0)
    pltpu.sync_copy(own_v, owner_hbm.at[pl.ds(lo, CH)])
    plsc.subcore_barrier()

    descs = []
    for g in range(NGRP):
        d = pltpu.make_async_copy(
            owner_hbm.at[lidx_v.at[pl.ds(g * GRP, GRP)]],
            o_v.at[pl.ds(g * GRP, GRP)], sem_o)
        d.start()
        descs.append(d)
    for d in descs:
        d.wait()

    zero = jnp.zeros((L,), jnp.int32)

    def zinit(i, c):
        ho[pl.ds(i * L, L)] = zero
        hlidx[pl.ds(i * L, L)] = zero
        return c

    lax.fori_loop(0, NVH, zinit, 0)

    def comp(i, nh):
        base = i * L
        o16 = o_v[pl.ds(base, L)]
        m = o16 >= 0
        plsc.store_compressed(hslot.at[pl.ds(nh, L)], iota + base, mask=m)
        plsc.store_compressed(ho.at[pl.ds(nh, L)], o16, mask=m)
        plsc.store_compressed(hlidx.at[pl.ds(nh, L)],
                              lidx_v[pl.ds(base, L)], mask=m)
        return nh + plsc.all_reduce_population_count(m)[0]

    nh = lax.fori_loop(0, LPW // L, comp, 0)

    rows16 = [(blk * L + iota) * NF for blk in range(NBLK)]

    def plane(c, par):
        csplat = jnp.full((L,), 0, jnp.int32) + c
        for blk in range(NBLK):
            a = jnp.zeros((L,), jnp.float32)
            for f in range(NF):
                a = a + plsc.load_gather(gbuf.at[par], [rows16[blk] + f])
            plsc.store_scatter(acc, [blk * L + iota, csplat], a)

    def outer(i, cr):
        for par in range(2):
            c = 2 * i + par
            wait_plane(par)
            plane(c, par)

            @pl.when(c + 2 < D)
            def _():
                start_plane(c + 2, par)
        return cr

    lax.fori_loop(0, D // 2, outer, 0)

    nch = (nh + GRPH - 1) // GRPH

    def corr(cc, cr):
        off = cc * GRPH
        hdescs = []
        for c in range(D):
            dv = pltpu.make_async_copy(
                valt_hbm.at[c].at[ho.at[pl.ds(off, GRPH)]], hv.at[c], sem_h)
            dm = pltpu.make_async_copy(
                memt_hbm.at[c].at[hlidx.at[pl.ds(off, GRPH)]], hm.at[c], sem_h)
            dv.start()
            dm.start()
            hdescs.append(dv)
            hdescs.append(dm)
        for d in hdescs:
            d.wait()
        for c in range(D):
            csplat = jnp.full((L,), c, jnp.int32)

            def blk(q, cr2):
                base = q * L
                s16 = hslot[pl.ds(off + base, L)]
                m = (off + base + iota) < nh
                b16 = s16 // NF
                dlt = hv[c, pl.ds(base, L)] - hm[c, pl.ds(base, L)]
                plsc.addupdate_scatter(acc, [b16, csplat], dlt, mask=m)
                return cr2

            lax.fori_loop(0, GRPH // L, blk, 0)
        return cr

    lax.fori_loop(0, nch, corr, 0)
    pltpu.sync_copy(acc, out_hbm.at[pl.ds(w * BPW, BPW)])


def kernel(mem, idx, val, lookup_idx):
    return _fused(mem.T, idx, val.T, lookup_idx)


# speedup vs baseline: 4.9827x; 4.9827x over previous
import functools

import jax
import jax.numpy as jnp
from jax import lax
from jax.experimental import pallas as pl
from jax.experimental.pallas import tpu as pltpu
from jax.experimental.pallas import tpu_sc as plsc

M = 1_000_000
D = 32
B_UPD = 16_384
B_LKP = 4_096
NF = 26

NC, NS, L = 2, 16, 16
NW = NC * NS
CH = 31_264
M_PAD = NW * CH
BPW = B_LKP // NW
LPW = BPW * NF
GRP = 4 * NF
NGRP = LPW // GRP
SUP = 4
SGR = SUP * GRP
NSG = LPW // SGR
NB = 2
GRPH = 112
NVH = (LPW + GRPH) // L

_mesh = plsc.VectorSubcoreMesh(core_axis_name="c", subcore_axis_name="s")
_params = pltpu.CompilerParams(needs_layout_passes=False,
                               use_tc_tiling_on_sc=False)


def _wid():
    return lax.axis_index("s") * NC + lax.axis_index("c")


@functools.partial(
    pl.kernel,
    out_type=jax.ShapeDtypeStruct((M_PAD,), jnp.int32),
    mesh=_mesh,
    scratch_types=[
        pltpu.VMEM((B_UPD,), jnp.int32),
        pltpu.VMEM((CH,), jnp.int32),
    ],
    compiler_params=_params,
)
def _build_owner(idx_hbm, owner_hbm, idx_v, own_v):
    w = _wid()
    lo = w * CH
    pltpu.sync_copy(idx_hbm, idx_v)
    neg = jnp.full((L,), -1, jnp.int32)

    def mset(i, c):
        own_v[pl.ds(i * L, L)] = neg
        return c

    lax.fori_loop(0, CH // L, mset, 0)

    iota = lax.iota(jnp.int32, L)

    def scan(j, c):
        base = j * L
        rel = idx_v[pl.ds(base, L)] - lo
        m = (rel >= 0) & (rel < CH)
        plsc.store_scatter(own_v, [rel], iota + base, mask=m)
        return c

    lax.fori_loop(0, B_UPD // L, scan, 0)
    pltpu.sync_copy(own_v, owner_hbm.at[pl.ds(lo, CH)])


@functools.partial(
    pl.kernel,
    out_type=jax.ShapeDtypeStruct((B_LKP, D), jnp.float32),
    mesh=_mesh,
    scratch_types=[
        pltpu.VMEM((LPW,), jnp.int32),
        pltpu.VMEM((LPW + L,), jnp.int32),
        pltpu.VMEM((NB, SGR, D), jnp.float32),
        pltpu.VMEM((BPW, D), jnp.float32),
        pltpu.VMEM((NVH * L,), jnp.int32),
        pltpu.VMEM((NVH * L,), jnp.int32),
        pltpu.VMEM((NVH * L,), jnp.int32),
        pltpu.VMEM((GRPH, D), jnp.float32),
        pltpu.VMEM((GRPH, D), jnp.float32),
        pltpu.SemaphoreType.DMA,
        pltpu.SemaphoreType.DMA,
        pltpu.SemaphoreType.DMA,
    ],
    compiler_params=_params,
)
def _lookup(mem_hbm, val_hbm, owner_hbm, lidx_hbm, out_hbm,
            lidx_v, o_v, rbuf, acc, hslot, ho, hlidx, vrow, mrow,
            sem_o, sem_m, sem_h):
    w = _wid()
    pltpu.sync_copy(lidx_hbm.at[pl.ds(w * LPW, LPW)], lidx_v)

    def start_sg(sg, par):
        for k in range(SUP):
            off = sg * SGR + k * GRP
            pltpu.make_async_copy(
                mem_hbm.at[lidx_v.at[pl.ds(off, GRP)]],
                rbuf.at[par, pl.ds(k * GRP, GRP)], sem_m).start()

    def wait_sg(par):
        for k in range(SUP):
            pltpu.make_async_copy(
                mem_hbm.at[lidx_v.at[pl.ds(0, GRP)]],
                rbuf.at[par, pl.ds(k * GRP, GRP)], sem_m).wait()

    start_sg(0, 0)
    start_sg(1, 1)

    descs = []
    for g in range(NGRP):
        d = pltpu.make_async_copy(
            owner_hbm.at[lidx_v.at[pl.ds(g * GRP, GRP)]],
            o_v.at[pl.ds(g * GRP, GRP)], sem_o)
        d.start()
        descs.append(d)
    for d in descs:
        d.wait()

    zero = jnp.zeros((L,), jnp.int32)

    def zinit(i, c):
        ho[pl.ds(i * L, L)] = zero
        hlidx[pl.ds(i * L, L)] = zero
        return c

    lax.fori_loop(0, NVH, zinit, 0)

    iota = lax.iota(jnp.int32, L)

    def comp(i, nh):
        base = i * L
        o16 = o_v[pl.ds(base, L)]
        m = o16 >= 0
        plsc.store_compressed(hslot.at[pl.ds(nh, L)], iota + base, mask=m)
        plsc.store_compressed(ho.at[pl.ds(nh, L)], o16, mask=m)
        plsc.store_compressed(hlidx.at[pl.ds(nh, L)],
                              lidx_v[pl.ds(base, L)], mask=m)
        return nh + plsc.all_reduce_population_count(m)[0]

    nh = lax.fori_loop(0, LPW // L, comp, 0)

    def outer(i, c):
        for par in range(NB):
            sg = NB * i + par
            wait_sg(par)
            for bl in range(SGR // NF):
                alo = jnp.zeros((L,), jnp.float32)
                ahi = jnp.zeros((L,), jnp.float32)
                for f in range(NF):
                    r = bl * NF + f
                    alo = alo + rbuf[par, r, pl.ds(0, L)]
                    ahi = ahi + rbuf[par, r, pl.ds(L, L)]
                b = sg * (SGR // NF) + bl
                acc[b, pl.ds(0, L)] = alo
                acc[b, pl.ds(L, L)] = ahi

            @pl.when(sg + NB < NSG)
            def _():
                start_sg(sg + NB, par)
        return c

    lax.fori_loop(0, NSG // NB, outer, 0)

    nch = (nh + GRPH - 1) // GRPH

    def corr(cc, c):
        off = cc * GRPH
        dv = pltpu.make_async_copy(
            val_hbm.at[ho.at[pl.ds(off, GRPH)]], vrow, sem_h)
        dm = pltpu.make_async_copy(
            mem_hbm.at[hlidx.at[pl.ds(off, GRPH)]], mrow, sem_h)
        dv.start()
        dm.start()
        dv.wait()
        dm.wait()

        def blk(q, c2):
            base = q * L
            s16 = hslot[pl.ds(off + base, L)]
            for kk in range(L):
                @pl.when(off + base + kk < nh)
                def _():
                    b = s16[kk] // NF
                    r = base + kk
                    acc[b, pl.ds(0, L)] = (acc[b, pl.ds(0, L)]
                                           + vrow[r, pl.ds(0, L)]
                                           - mrow[r, pl.ds(0, L)])
                    acc[b, pl.ds(L, L)] = (acc[b, pl.ds(L, L)]
                                           + vrow[r, pl.ds(L, L)]
                                           - mrow[r, pl.ds(L, L)])
            return c2

        lax.fori_loop(0, GRPH // L, blk, 0)
        return c

    lax.fori_loop(0, nch, corr, 0)
    pltpu.sync_copy(acc, out_hbm.at[pl.ds(w * BPW, BPW)])


def kernel(mem, idx, val, lookup_idx):
    owner = _build_owner(idx)
    return _lookup(mem, val, owner, lookup_idx.reshape(-1))
